# baseline (speedup 1.0000x reference)
import functools

import jax
import jax.numpy as jnp
import numpy as np
from jax import lax
from jax.experimental import pallas as pl
from jax.experimental.pallas import tpu as pltpu
from jax.experimental.pallas import tpu_sc as plsc

EMBED = 768
DPE = EMBED // 4
MAX_SEQ = 24
BASE_GSD = 10.0

L = 16
NC = 2
NS = 16
NW = NC * NS

B_, H_, W_, T_, C_ = 4, 16, 16, 12, 3
ROWS = B_ * H_ * W_ * T_ * C_
RPH = T_ * C_
HW_PER_W = (B_ * H_ * W_) // NW
HW_PER_CHUNK = 2
RPC = RPH * HW_PER_CHUNK
CHUNKS = HW_PER_W // HW_PER_CHUNK
SEG = DPE // L


def _sincos_1d(pos, dim):
    omega = jnp.arange(dim // 2, dtype=jnp.float32) / (dim / 2.0)
    omega = 1.0 / (10000.0 ** omega)
    out = pos.astype(jnp.float32)[:, None] * omega[None, :]
    return jnp.concatenate([jnp.sin(out), jnp.cos(out)], axis=-1)


def _month_table(dim):
    angles = jnp.arange(12, dtype=jnp.float32) * (2.0 * np.pi / 12.0)
    sin_t = jnp.repeat(jnp.sin(angles)[:, None], dim // 2, axis=1)
    cos_t = jnp.repeat(jnp.cos(angles)[:, None], dim // 2, axis=1)
    return jnp.concatenate([sin_t, cos_t], axis=-1)


def _sincos_2d_res(grid_size, res, dim):
    g = jnp.arange(grid_size, dtype=jnp.float32)
    gw, gh = jnp.meshgrid(g, g, indexing="xy")
    grid = jnp.stack([gw, gh], axis=0)[None] * res[:, None, None, None]

    def emb(p):
        p = p.reshape(p.shape[0], -1)
        omega = jnp.arange(dim // 4, dtype=jnp.float32) / (dim / 4.0)
        omega = 1.0 / (10000.0 ** omega)
        out = p[..., None] * omega
        return jnp.concatenate([jnp.sin(out), jnp.cos(out)], axis=-1)

    return jnp.concatenate([emb(grid[:, 1]), emb(grid[:, 0])], axis=-1)


def _sc_body(tok_hbm, ch_hbm, pos_hbm, mt_hbm, mon_hbm, sp_hbm, out_hbm,
             ch_v, pos_v, memb_v, sp_v, midx_v, buf, gsem):
    cid = lax.axis_index("c")
    sid = lax.axis_index("s")
    wid = sid * NC + cid
    b = wid // (NW // B_)
    base = wid * HW_PER_W

    pltpu.sync_copy(ch_hbm, ch_v)
    pltpu.sync_copy(pos_hbm, pos_v)
    pltpu.sync_copy(sp_hbm.at[pl.ds(base, HW_PER_W)], sp_v)
    pltpu.sync_copy(mon_hbm.at[pl.ds(b * 16, 16)], midx_v)
    pltpu.async_copy(mt_hbm.at[midx_v], memb_v, gsem).wait()

    def compute(j, bi_buf):
        for s in range(HW_PER_CHUNK):
            jj = j * HW_PER_CHUNK + s
            r0 = s * RPH
            spv = [sp_v[jj, pl.ds(k * L, L)] for k in range(SEG)]

            def t_body(t, _, r0=r0, spv=spv):
                pv = [pos_v[t, pl.ds(k * L, L)] for k in range(SEG)]
                mv = [memb_v[t, pl.ds(k * L, L)] for k in range(SEG)]

                def c_body(c, __):
                    r = r0 + t * C_ + c
                    for k in range(SEG):
                        sl = pl.ds(k * L, L)
                        bi_buf[r, sl] = bi_buf[r, sl] + ch_v[c, sl]
                    for k in range(SEG):
                        sl = pl.ds(DPE + k * L, L)
                        bi_buf[r, sl] = bi_buf[r, sl] + pv[k]
                    for k in range(SEG):
                        sl = pl.ds(2 * DPE + k * L, L)
                        bi_buf[r, sl] = bi_buf[r, sl] + mv[k]
                    for k in range(SEG):
                        sl = pl.ds(3 * DPE + k * L, L)
                        bi_buf[r, sl] = bi_buf[r, sl] + spv[k]
                    return 0

                lax.fori_loop(0, C_, c_body, 0)
                return 0

            lax.fori_loop(0, T_, t_body, 0)

    def chunk(j, _):
        row0 = base * RPH + j * RPC
        pltpu.sync_copy(tok_hbm.at[pl.ds(row0, RPC)], buf)
        compute(j, buf)
        pltpu.sync_copy(buf, out_hbm.at[pl.ds(row0, RPC)])
        return 0

    lax.fori_loop(0, CHUNKS, chunk, 0)


@functools.partial(jax.jit)
def _sc_call(tok2, ch, pos, mt, months_p, sp_flat):
    mesh = plsc.VectorSubcoreMesh(core_axis_name="c", subcore_axis_name="s")
    return pl.kernel(
        _sc_body,
        out_type=jax.ShapeDtypeStruct((ROWS, EMBED), jnp.float32),
        mesh=mesh,
        scratch_types=[
            pltpu.VMEM((C_, DPE), jnp.float32),
            pltpu.VMEM((T_, DPE), jnp.float32),
            pltpu.VMEM((16, 256), jnp.float32),
            pltpu.VMEM((HW_PER_W, DPE), jnp.float32),
            pltpu.VMEM((16,), jnp.int32),
            pltpu.VMEM((RPC, EMBED), jnp.float32),
            pltpu.SemaphoreType.DMA,
        ],
    )(tok2, ch, pos, mt, months_p, sp_flat)


def kernel(per_modality_input_tokens, timestamps, channel_emb, patch_size, input_res):
    tokens = per_modality_input_tokens
    b, h, w, t, c, d = tokens.shape
    dpe = d // 4
    pos = _sincos_1d(jnp.arange(MAX_SEQ), dpe)[:t]
    mt = _month_table(dpe)
    mt = jnp.pad(mt, ((0, 0), (0, 256 - dpe)))
    months = timestamps[:, :, 1].astype(jnp.int32)
    months_p = jnp.pad(months, ((0, 0), (0, 16 - t))).reshape(-1)
    gsd_ratio = input_res * patch_size / BASE_GSD
    res = jnp.ones((b,), dtype=jnp.float32) * gsd_ratio
    sp = _sincos_2d_res(h, res, dpe)
    sp_flat = sp.reshape(b * h * w, dpe).astype(jnp.float32)
    tok2 = tokens.reshape(ROWS, d)
    out = _sc_call(tok2, channel_emb.astype(jnp.float32), pos, mt, months_p, sp_flat)
    return out.reshape(tokens.shape)


# speedup vs baseline: 1.0528x; 1.0528x over previous
import functools

import jax
import jax.numpy as jnp
import numpy as np
from jax import lax
from jax.experimental import pallas as pl
from jax.experimental.pallas import tpu as pltpu
from jax.experimental.pallas import tpu_sc as plsc

EMBED = 768
DPE = EMBED // 4
MAX_SEQ = 24
BASE_GSD = 10.0

L = 16
NC = 2
NS = 16
NW = NC * NS

B_, H_, W_, T_, C_ = 4, 16, 16, 12, 3
ROWS = B_ * H_ * W_ * T_ * C_
RPH = T_ * C_
HW_PER_W = (B_ * H_ * W_) // NW
HW_PER_CHUNK = 2
RPC = RPH * HW_PER_CHUNK
CHUNKS = HW_PER_W // HW_PER_CHUNK
SEG = DPE // L


def _sincos_1d(pos, dim):
    omega = jnp.arange(dim // 2, dtype=jnp.float32) / (dim / 2.0)
    omega = 1.0 / (10000.0 ** omega)
    out = pos.astype(jnp.float32)[:, None] * omega[None, :]
    return jnp.concatenate([jnp.sin(out), jnp.cos(out)], axis=-1)


def _month_table(dim):
    angles = jnp.arange(12, dtype=jnp.float32) * (2.0 * np.pi / 12.0)
    sin_t = jnp.repeat(jnp.sin(angles)[:, None], dim // 2, axis=1)
    cos_t = jnp.repeat(jnp.cos(angles)[:, None], dim // 2, axis=1)
    return jnp.concatenate([sin_t, cos_t], axis=-1)


def _sincos_2d_res(grid_size, res, dim):
    g = jnp.arange(grid_size, dtype=jnp.float32)
    gw, gh = jnp.meshgrid(g, g, indexing="xy")
    grid = jnp.stack([gw, gh], axis=0)[None] * res[:, None, None, None]

    def emb(p):
        p = p.reshape(p.shape[0], -1)
        omega = jnp.arange(dim // 4, dtype=jnp.float32) / (dim / 4.0)
        omega = 1.0 / (10000.0 ** omega)
        out = p[..., None] * omega
        return jnp.concatenate([jnp.sin(out), jnp.cos(out)], axis=-1)

    return jnp.concatenate([emb(grid[:, 1]), emb(grid[:, 0])], axis=-1)


def _sc_body(tok_hbm, ch_hbm, pos_hbm, mt_hbm, mon_hbm, sp_hbm, out_hbm,
             ch_v, pos_v, memb_v, sp_v, midx_v, buf0, buf1,
             isem0, isem1, osem0, osem1, gsem):
    bufs = (buf0, buf1)
    isems = (isem0, isem1)
    osems = (osem0, osem1)
    cid = lax.axis_index("c")
    sid = lax.axis_index("s")
    wid = sid * NC + cid
    b = wid // (NW // B_)
    base = wid * HW_PER_W

    pltpu.sync_copy(ch_hbm, ch_v)
    pltpu.sync_copy(pos_hbm, pos_v)
    pltpu.sync_copy(sp_hbm.at[pl.ds(base, HW_PER_W)], sp_v)
    pltpu.sync_copy(mon_hbm.at[pl.ds(b * 16, 16)], midx_v)
    pltpu.async_copy(mt_hbm.at[midx_v], memb_v, gsem).wait()

    def compute_dyn(j, bi_buf):
        for s in range(HW_PER_CHUNK):
            jj = j * HW_PER_CHUNK + s
            r0 = s * RPH
            spv = [sp_v[jj, pl.ds(k * L, L)] for k in range(SEG)]

            def t_body(t, _, r0=r0, spv=spv):
                pv = [pos_v[t, pl.ds(k * L, L)] for k in range(SEG)]
                mv = [memb_v[t, pl.ds(k * L, L)] for k in range(SEG)]

                def c_body(c, __):
                    r = r0 + t * C_ + c
                    for k in range(SEG):
                        sl = pl.ds(k * L, L)
                        bi_buf[r, sl] = bi_buf[r, sl] + ch_v[c, sl]
                    for k in range(SEG):
                        sl = pl.ds(DPE + k * L, L)
                        bi_buf[r, sl] = bi_buf[r, sl] + pv[k]
                    for k in range(SEG):
                        sl = pl.ds(2 * DPE + k * L, L)
                        bi_buf[r, sl] = bi_buf[r, sl] + mv[k]
                    for k in range(SEG):
                        sl = pl.ds(3 * DPE + k * L, L)
                        bi_buf[r, sl] = bi_buf[r, sl] + spv[k]
                    return 0

                lax.fori_loop(0, C_, c_body, 0)
                return 0

            lax.fori_loop(0, T_, t_body, 0)

    def tok_slice(j):
        return tok_hbm.at[pl.ds(base * RPH + j * RPC, RPC)]

    def out_slice(j):
        return out_hbm.at[pl.ds(base * RPH + j * RPC, RPC)]

    pltpu.async_copy(tok_slice(0), bufs[0], isems[0])
    pltpu.async_copy(tok_slice(1), bufs[1], isems[1])

    def group(g, _):
        for s in range(2):
            j = g * 2 + s
            pltpu.make_async_copy(tok_slice(j), bufs[s], isems[s]).wait()
            compute_dyn(j, bufs[s])
            pltpu.async_copy(bufs[s], out_slice(j), osems[s])

        @pl.when(g < CHUNKS // 2 - 1)
        def _():
            for s in range(2):
                j = g * 2 + s
                pltpu.make_async_copy(bufs[s], out_slice(j), osems[s]).wait()
                pltpu.async_copy(tok_slice(j + 2), bufs[s], isems[s])

        return 0

    lax.fori_loop(0, CHUNKS // 2, group, 0)
    pltpu.make_async_copy(bufs[0], out_slice(CHUNKS - 2), osems[0]).wait()
    pltpu.make_async_copy(bufs[1], out_slice(CHUNKS - 1), osems[1]).wait()


@functools.partial(jax.jit)
def _sc_call(tok2, ch, pos, mt, months_p, sp_flat):
    mesh = plsc.VectorSubcoreMesh(core_axis_name="c", subcore_axis_name="s")
    return pl.kernel(
        _sc_body,
        out_type=jax.ShapeDtypeStruct((ROWS, EMBED), jnp.float32),
        mesh=mesh,
        scratch_types=[
            pltpu.VMEM((C_, DPE), jnp.float32),
            pltpu.VMEM((T_, DPE), jnp.float32),
            pltpu.VMEM((16, 256), jnp.float32),
            pltpu.VMEM((HW_PER_W, DPE), jnp.float32),
            pltpu.VMEM((16,), jnp.int32),
            pltpu.VMEM((RPC, EMBED), jnp.float32),
            pltpu.VMEM((RPC, EMBED), jnp.float32),
            pltpu.SemaphoreType.DMA,
            pltpu.SemaphoreType.DMA,
            pltpu.SemaphoreType.DMA,
            pltpu.SemaphoreType.DMA,
            pltpu.SemaphoreType.DMA,
        ],
    )(tok2, ch, pos, mt, months_p, sp_flat)


def kernel(per_modality_input_tokens, timestamps, channel_emb, patch_size, input_res):
    tokens = per_modality_input_tokens
    b, h, w, t, c, d = tokens.shape
    dpe = d // 4
    pos = _sincos_1d(jnp.arange(MAX_SEQ), dpe)[:t]
    mt = _month_table(dpe)
    mt = jnp.pad(mt, ((0, 0), (0, 256 - dpe)))
    months = timestamps[:, :, 1].astype(jnp.int32)
    months_p = jnp.pad(months, ((0, 0), (0, 16 - t))).reshape(-1)
    gsd_ratio = input_res * patch_size / BASE_GSD
    res = jnp.ones((b,), dtype=jnp.float32) * gsd_ratio
    sp = _sincos_2d_res(h, res, dpe)
    sp_flat = sp.reshape(b * h * w, dpe).astype(jnp.float32)
    tok2 = tokens.reshape(ROWS, d)
    out = _sc_call(tok2, channel_emb.astype(jnp.float32), pos, mt, months_p, sp_flat)
    return out.reshape(tokens.shape)


# speedup vs baseline: 1.1118x; 1.0560x over previous
import functools

import jax
import jax.numpy as jnp
import numpy as np
from jax import lax
from jax.experimental import pallas as pl
from jax.experimental.pallas import tpu as pltpu
from jax.experimental.pallas import tpu_sc as plsc

EMBED = 768
DPE = EMBED // 4
MAX_SEQ = 24
BASE_GSD = 10.0

L = 16
NC = 2
NS = 16
NW = NC * NS

B_, H_, W_, T_, C_ = 4, 16, 16, 12, 3
ROWS = B_ * H_ * W_ * T_ * C_
RPH = T_ * C_
HW_PER_W = (B_ * H_ * W_) // NW
HW_PER_CHUNK = 2
RPC = RPH * HW_PER_CHUNK
CHUNKS = HW_PER_W // HW_PER_CHUNK
SEG = DPE // L


def _sincos_1d(pos, dim):
    omega = jnp.arange(dim // 2, dtype=jnp.float32) / (dim / 2.0)
    omega = 1.0 / (10000.0 ** omega)
    out = pos.astype(jnp.float32)[:, None] * omega[None, :]
    return jnp.concatenate([jnp.sin(out), jnp.cos(out)], axis=-1)


def _month_table(dim):
    angles = jnp.arange(12, dtype=jnp.float32) * (2.0 * np.pi / 12.0)
    sin_t = jnp.repeat(jnp.sin(angles)[:, None], dim // 2, axis=1)
    cos_t = jnp.repeat(jnp.cos(angles)[:, None], dim // 2, axis=1)
    return jnp.concatenate([sin_t, cos_t], axis=-1)


def _sincos_2d_res(grid_size, res, dim):
    g = jnp.arange(grid_size, dtype=jnp.float32)
    gw, gh = jnp.meshgrid(g, g, indexing="xy")
    grid = jnp.stack([gw, gh], axis=0)[None] * res[:, None, None, None]

    def emb(p):
        p = p.reshape(p.shape[0], -1)
        omega = jnp.arange(dim // 4, dtype=jnp.float32) / (dim / 4.0)
        omega = 1.0 / (10000.0 ** omega)
        out = p[..., None] * omega
        return jnp.concatenate([jnp.sin(out), jnp.cos(out)], axis=-1)

    return jnp.concatenate([emb(grid[:, 1]), emb(grid[:, 0])], axis=-1)


def _sc_body(tok_hbm, ch_hbm, pos_hbm, mt_hbm, mon_hbm, sp_hbm, out_hbm,
             ch_v, pos_v, memb_v, sp_v, midx_v, buf0, buf1,
             isem0, isem1, osem0, osem1, gsem):
    bufs = (buf0, buf1)
    isems = (isem0, isem1)
    osems = (osem0, osem1)
    cid = lax.axis_index("c")
    sid = lax.axis_index("s")
    wid = sid * NC + cid
    b = wid // (NW // B_)
    base = wid * HW_PER_W

    pltpu.sync_copy(ch_hbm, ch_v)
    pltpu.sync_copy(pos_hbm, pos_v)
    pltpu.sync_copy(sp_hbm.at[pl.ds(base, HW_PER_W)], sp_v)
    pltpu.sync_copy(mon_hbm.at[pl.ds(b * 16, 16)], midx_v)
    pltpu.async_copy(mt_hbm.at[midx_v], memb_v, gsem).wait()

    def compute_dyn(j, bi_buf):
        for s in range(HW_PER_CHUNK):
            jj = j * HW_PER_CHUNK + s
            r0 = s * RPH
            spv = [sp_v[jj, pl.ds(k * L, L)] for k in range(SEG)]

            @plsc.parallel_loop(0, RPH)
            def row_body(rr, r0=r0, spv=spv):
                t = rr // C_
                c = rr - t * C_
                r = r0 + rr
                for k in range(SEG):
                    sl = pl.ds(k * L, L)
                    bi_buf[r, sl] = bi_buf[r, sl] + ch_v[c, sl]
                for k in range(SEG):
                    sl = pl.ds(k * L, L)
                    bi_buf[r, pl.ds(DPE + k * L, L)] = (
                        bi_buf[r, pl.ds(DPE + k * L, L)] + pos_v[t, sl])
                for k in range(SEG):
                    sl = pl.ds(k * L, L)
                    bi_buf[r, pl.ds(2 * DPE + k * L, L)] = (
                        bi_buf[r, pl.ds(2 * DPE + k * L, L)] + memb_v[t, sl])
                for k in range(SEG):
                    bi_buf[r, pl.ds(3 * DPE + k * L, L)] = (
                        bi_buf[r, pl.ds(3 * DPE + k * L, L)] + spv[k])

    def tok_slice(j):
        return tok_hbm.at[pl.ds(base * RPH + j * RPC, RPC)]

    def out_slice(j):
        return out_hbm.at[pl.ds(base * RPH + j * RPC, RPC)]

    pltpu.async_copy(tok_slice(0), bufs[0], isems[0])
    pltpu.async_copy(tok_slice(1), bufs[1], isems[1])

    def group(g, _):
        for s in range(2):
            j = g * 2 + s
            pltpu.make_async_copy(tok_slice(j), bufs[s], isems[s]).wait()
            compute_dyn(j, bufs[s])
            pltpu.async_copy(bufs[s], out_slice(j), osems[s])

        @pl.when(g < CHUNKS // 2 - 1)
        def _():
            for s in range(2):
                j = g * 2 + s
                pltpu.make_async_copy(bufs[s], out_slice(j), osems[s]).wait()
                pltpu.async_copy(tok_slice(j + 2), bufs[s], isems[s])

        return 0

    lax.fori_loop(0, CHUNKS // 2, group, 0)
    pltpu.make_async_copy(bufs[0], out_slice(CHUNKS - 2), osems[0]).wait()
    pltpu.make_async_copy(bufs[1], out_slice(CHUNKS - 1), osems[1]).wait()


@functools.partial(jax.jit)
def _sc_call(tok2, ch, pos, mt, months_p, sp_flat):
    mesh = plsc.VectorSubcoreMesh(core_axis_name="c", subcore_axis_name="s")
    return pl.kernel(
        _sc_body,
        out_type=jax.ShapeDtypeStruct((ROWS, EMBED), jnp.float32),
        mesh=mesh,
        scratch_types=[
            pltpu.VMEM((C_, DPE), jnp.float32),
            pltpu.VMEM((T_, DPE), jnp.float32),
            pltpu.VMEM((16, 256), jnp.float32),
            pltpu.VMEM((HW_PER_W, DPE), jnp.float32),
            pltpu.VMEM((16,), jnp.int32),
            pltpu.VMEM((RPC, EMBED), jnp.float32),
            pltpu.VMEM((RPC, EMBED), jnp.float32),
            pltpu.SemaphoreType.DMA,
            pltpu.SemaphoreType.DMA,
            pltpu.SemaphoreType.DMA,
            pltpu.SemaphoreType.DMA,
            pltpu.SemaphoreType.DMA,
        ],
    )(tok2, ch, pos, mt, months_p, sp_flat)


def kernel(per_modality_input_tokens, timestamps, channel_emb, patch_size, input_res):
    tokens = per_modality_input_tokens
    b, h, w, t, c, d = tokens.shape
    dpe = d // 4
    pos = _sincos_1d(jnp.arange(MAX_SEQ), dpe)[:t]
    mt = _month_table(dpe)
    mt = jnp.pad(mt, ((0, 0), (0, 256 - dpe)))
    months = timestamps[:, :, 1].astype(jnp.int32)
    months_p = jnp.pad(months, ((0, 0), (0, 16 - t))).reshape(-1)
    gsd_ratio = input_res * patch_size / BASE_GSD
    res = jnp.ones((b,), dtype=jnp.float32) * gsd_ratio
    sp = _sincos_2d_res(h, res, dpe)
    sp_flat = sp.reshape(b * h * w, dpe).astype(jnp.float32)
    tok2 = tokens.reshape(ROWS, d)
    out = _sc_call(tok2, channel_emb.astype(jnp.float32), pos, mt, months_p, sp_flat)
    return out.reshape(tokens.shape)


# speedup vs baseline: 6.0214x; 5.4159x over previous
import functools

import jax
import jax.numpy as jnp
import numpy as np
from jax import lax
from jax.experimental import pallas as pl
from jax.experimental.pallas import tpu as pltpu
from jax.experimental.pallas import tpu_sc as plsc

EMBED = 768
DPE = EMBED // 4
MAX_SEQ = 24
BASE_GSD = 10.0

L = 16
NC = 2
NS = 16
NW = NC * NS

B_, H_, W_, T_, C_ = 4, 16, 16, 12, 3
ROWS = B_ * H_ * W_ * T_ * C_
RPC = C_ * W_
SLABS_PER_W = 2
CHUNKS = SLABS_PER_W * T_
SEG = DPE // L


def _sincos_1d(pos, dim):
    omega = jnp.arange(dim // 2, dtype=jnp.float32) / (dim / 2.0)
    omega = 1.0 / (10000.0 ** omega)
    out = pos.astype(jnp.float32)[:, None] * omega[None, :]
    return jnp.concatenate([jnp.sin(out), jnp.cos(out)], axis=-1)


def _month_table(dim):
    angles = jnp.arange(12, dtype=jnp.float32) * (2.0 * np.pi / 12.0)
    sin_t = jnp.repeat(jnp.sin(angles)[:, None], dim // 2, axis=1)
    cos_t = jnp.repeat(jnp.cos(angles)[:, None], dim // 2, axis=1)
    return jnp.concatenate([sin_t, cos_t], axis=-1)


def _sincos_2d_res(grid_size, res, dim):
    g = jnp.arange(grid_size, dtype=jnp.float32)
    gw, gh = jnp.meshgrid(g, g, indexing="xy")
    grid = jnp.stack([gw, gh], axis=0)[None] * res[:, None, None, None]

    def emb(p):
        p = p.reshape(p.shape[0], -1)
        omega = jnp.arange(dim // 4, dtype=jnp.float32) / (dim / 4.0)
        omega = 1.0 / (10000.0 ** omega)
        out = p[..., None] * omega
        return jnp.concatenate([jnp.sin(out), jnp.cos(out)], axis=-1)

    return jnp.concatenate([emb(grid[:, 1]), emb(grid[:, 0])], axis=-1)


def _sc_body(tok_hbm, ch_hbm, pos_hbm, mt_hbm, mon_hbm, sp_hbm, out_hbm,
             ch_v, pos_v, memb_v, sp_v, midx_v, buf0, buf1, buf2,
             isem0, isem1, isem2, osem0, osem1, osem2, gsem):
    bufs = (buf0, buf1, buf2)
    isems = (isem0, isem1, isem2)
    osems = (osem0, osem1, osem2)
    cid = lax.axis_index("c")
    sid = lax.axis_index("s")
    wid = sid * NC + cid
    b = wid // (NW // B_)
    base_row = wid * (SLABS_PER_W * T_ * RPC)

    def tok_slice(q):
        return tok_hbm.at[pl.ds(base_row + q * RPC, RPC)]

    def out_slice(q):
        return out_hbm.at[pl.ds(base_row + q * RPC, RPC)]

    pltpu.async_copy(tok_slice(0), bufs[0], isems[0])

    pltpu.sync_copy(ch_hbm, ch_v)
    pltpu.sync_copy(pos_hbm, pos_v)
    pltpu.sync_copy(sp_hbm.at[pl.ds(wid * (SLABS_PER_W * W_), SLABS_PER_W * W_)], sp_v)
    pltpu.sync_copy(mon_hbm.at[pl.ds(b * 16, 16)], midx_v)
    pltpu.async_copy(mt_hbm.at[midx_v], memb_v, gsem).wait()

    def compute_dyn(q, bi_buf):
        sl = (q >= T_).astype(jnp.int32)
        t = q - sl * T_
        pv = [pos_v[t, pl.ds(k * L, L)] for k in range(SEG)]
        mv = [memb_v[t, pl.ds(k * L, L)] for k in range(SEG)]
        w_base = sl * W_

        @plsc.parallel_loop(0, RPC)
        def row_body(r):
            c = r >> 4
            w = r & 15
            sprow = w_base + w
            for k in range(SEG):
                sl_ = pl.ds(k * L, L)
                bi_buf[r, sl_] = bi_buf[r, sl_] + ch_v[c, sl_]
            for k in range(SEG):
                sl_ = pl.ds(k * L, L)
                bi_buf[r, pl.ds(DPE + k * L, L)] = (
                    bi_buf[r, pl.ds(DPE + k * L, L)] + pv[k])
            for k in range(SEG):
                bi_buf[r, pl.ds(2 * DPE + k * L, L)] = (
                    bi_buf[r, pl.ds(2 * DPE + k * L, L)] + mv[k])
            for k in range(SEG):
                sl_ = pl.ds(k * L, L)
                bi_buf[r, pl.ds(3 * DPE + k * L, L)] = (
                    bi_buf[r, pl.ds(3 * DPE + k * L, L)] + sp_v[sprow, sl_])

    def group(g, _):
        for s in range(3):
            q = g * 3 + s
            nb = (s + 1) % 3

            @pl.when(q + 1 < CHUNKS)
            def _():
                @pl.when(q - 2 >= 0)
                def _():
                    pltpu.make_async_copy(bufs[nb], out_slice(q - 2),
                                          osems[nb]).wait()
                pltpu.async_copy(tok_slice(q + 1), bufs[nb], isems[nb])

            pltpu.make_async_copy(tok_slice(q), bufs[s], isems[s]).wait()
            compute_dyn(q, bufs[s])
            pltpu.async_copy(bufs[s], out_slice(q), osems[s])
        return 0

    lax.fori_loop(0, CHUNKS // 3, group, 0)
    pltpu.make_async_copy(bufs[0], out_slice(CHUNKS - 3), osems[0]).wait()
    pltpu.make_async_copy(bufs[1], out_slice(CHUNKS - 2), osems[1]).wait()
    pltpu.make_async_copy(bufs[2], out_slice(CHUNKS - 1), osems[2]).wait()


@functools.partial(jax.jit)
def _sc_call(tok2, ch, pos, mt, months_p, sp_flat):
    mesh = plsc.VectorSubcoreMesh(core_axis_name="c", subcore_axis_name="s")
    return pl.kernel(
        _sc_body,
        out_type=jax.ShapeDtypeStruct((ROWS, EMBED), jnp.float32),
        mesh=mesh,
        scratch_types=[
            pltpu.VMEM((C_, DPE), jnp.float32),
            pltpu.VMEM((T_, DPE), jnp.float32),
            pltpu.VMEM((16, 256), jnp.float32),
            pltpu.VMEM((SLABS_PER_W * W_, DPE), jnp.float32),
            pltpu.VMEM((16,), jnp.int32),
            pltpu.VMEM((RPC, EMBED), jnp.float32),
            pltpu.VMEM((RPC, EMBED), jnp.float32),
            pltpu.VMEM((RPC, EMBED), jnp.float32),
            pltpu.SemaphoreType.DMA,
            pltpu.SemaphoreType.DMA,
            pltpu.SemaphoreType.DMA,
            pltpu.SemaphoreType.DMA,
            pltpu.SemaphoreType.DMA,
            pltpu.SemaphoreType.DMA,
            pltpu.SemaphoreType.DMA,
        ],
    )(tok2, ch, pos, mt, months_p, sp_flat)


def kernel(per_modality_input_tokens, timestamps, channel_emb, patch_size, input_res):
    tokens = per_modality_input_tokens
    b, h, w, t, c, d = tokens.shape
    dpe = d // 4
    pos = _sincos_1d(jnp.arange(MAX_SEQ), dpe)[:t]
    mt = _month_table(dpe)
    mt = jnp.pad(mt, ((0, 0), (0, 256 - dpe)))
    months = timestamps[:, :, 1].astype(jnp.int32)
    months_p = jnp.pad(months, ((0, 0), (0, 16 - t))).reshape(-1)
    gsd_ratio = input_res * patch_size / BASE_GSD
    res = jnp.ones((b,), dtype=jnp.float32) * gsd_ratio
    sp = _sincos_2d_res(h, res, dpe)
    sp_flat = sp.reshape(b * h * w, dpe).astype(jnp.float32)
    tok2 = tokens.transpose(0, 1, 3, 4, 2, 5).reshape(ROWS, d)
    out = _sc_call(tok2, channel_emb.astype(jnp.float32), pos, mt, months_p, sp_flat)
    out = out.reshape(b, h, t, c, w, d).transpose(0, 1, 4, 2, 3, 5)
    return out
